# baseline (speedup 1.0000x reference)
import jax
import jax.numpy as jnp
from jax.experimental import pallas as pl

_B, _C, _H, _W = 16, 96, 224, 224


def _copy_body(x_ref, o_ref):
    o_ref[...] = x_ref[...]


def _fill_body(_, o_ref):
    o_ref[...] = jnp.full((1, 2, _H, _W), 4.0, jnp.float32)


def kernel(x):
    y = pl.pallas_call(
        _copy_body,
        grid=(_B, 2),
        in_specs=[
            pl.BlockSpec(
                (pl.Element(1), pl.Element(47), pl.Element(_H), pl.Element(_W)),
                lambda i, j: (i, 2 + 47 * j, 0, 0),
            )
        ],
        out_specs=pl.BlockSpec(
            (pl.Element(1), pl.Element(47), pl.Element(_H), pl.Element(_W)),
            lambda i, j: (i, 2 + 47 * j, 0, 0),
        ),
        out_shape=jax.ShapeDtypeStruct((_B, _C, _H, _W), x.dtype),
    )(x)
    y = pl.pallas_call(
        _fill_body,
        grid=(_B,),
        in_specs=[pl.BlockSpec(memory_space=pl.ANY)],
        out_specs=pl.BlockSpec((1, 2, _H, _W), lambda i: (i, 0, 0, 0)),
        out_shape=jax.ShapeDtypeStruct((_B, _C, _H, _W), x.dtype),
        input_output_aliases={0: 0},
    )(y)
    return y


# speedup vs baseline: 1.0146x; 1.0146x over previous
import jax
import jax.numpy as jnp
from jax.experimental import pallas as pl

_B, _C, _H, _W = 16, 96, 224, 224


def _copy_body(x_ref, o_ref):
    o_ref[...] = x_ref[...]


def _fill_body(_, o_ref):
    o_ref[...] = jnp.full((_B, 2, _H, _W), 4.0, jnp.float32)


def kernel(x):
    y = pl.pallas_call(
        _copy_body,
        grid=(_B, 2),
        in_specs=[
            pl.BlockSpec(
                (pl.Element(1), pl.Element(47), pl.Element(_H), pl.Element(_W)),
                lambda i, j: (i, 2 + 47 * j, 0, 0),
            )
        ],
        out_specs=pl.BlockSpec(
            (pl.Element(1), pl.Element(47), pl.Element(_H), pl.Element(_W)),
            lambda i, j: (i, 2 + 47 * j, 0, 0),
        ),
        out_shape=jax.ShapeDtypeStruct((_B, _C, _H, _W), x.dtype),
    )(x)
    y = pl.pallas_call(
        _fill_body,
        grid=(1,),
        in_specs=[pl.BlockSpec(memory_space=pl.ANY)],
        out_specs=pl.BlockSpec((_B, 2, _H, _W), lambda i: (0, 0, 0, 0)),
        out_shape=jax.ShapeDtypeStruct((_B, _C, _H, _W), x.dtype),
        input_output_aliases={0: 0},
    )(y)
    return y
